# initial kernel scaffold (unmeasured)
import jax
import jax.numpy as jnp
from jax import lax
from jax.experimental import pallas as pl
from jax.experimental.pallas import tpu as pltpu

M = 4096
K = 2048
N = 4096
NC = 16
Nc = N // NC


def kernel(A, B):
    def body(a_hbm, b_ref, out_ref, a_vmem, send_buf, recv_buf,
             local_sem, send_sem, recv_sem, credit_sem):
        j = pl.program_id(0)
        my_x = lax.axis_index("x")
        my_y = lax.axis_index("y")
        neighbor = (my_x, 1 - my_y)

        @pl.when(j == 0)
        def _():
            cp = pltpu.make_async_copy(a_hbm, a_vmem, local_sem)
            cp.start()
            cp.wait()
            barrier = pltpu.get_barrier_semaphore()
            pl.semaphore_signal(
                barrier, inc=1,
                device_id=neighbor, device_id_type=pl.DeviceIdType.MESH,
            )
            pl.semaphore_wait(barrier, 1)

        send_buf[...] = jnp.dot(
            a_vmem[...], b_ref[...], preferred_element_type=jnp.float32
        )

        @pl.when(j > 0)
        def _():
            pl.semaphore_wait(credit_sem, 1)

        rdma = pltpu.make_async_remote_copy(
            src_ref=send_buf,
            dst_ref=recv_buf,
            send_sem=send_sem,
            recv_sem=recv_sem,
            device_id=neighbor,
            device_id_type=pl.DeviceIdType.MESH,
        )
        rdma.start()
        rdma.wait()

        out_ref[...] = send_buf[...] + recv_buf[...]
        pl.semaphore_signal(
            credit_sem, inc=1,
            device_id=neighbor, device_id_type=pl.DeviceIdType.MESH,
        )

        @pl.when(j == pl.num_programs(0) - 1)
        def _():
            pl.semaphore_wait(credit_sem, 1)

    return pl.pallas_call(
        body,
        grid=(NC,),
        out_shape=jax.ShapeDtypeStruct((M, N), jnp.float32),
        in_specs=[
            pl.BlockSpec(memory_space=pltpu.ANY),
            pl.BlockSpec((K, Nc), lambda j: (0, j)),
        ],
        out_specs=pl.BlockSpec((M, Nc), lambda j: (0, j)),
        scratch_shapes=[
            pltpu.VMEM((M, K), jnp.float32),
            pltpu.VMEM((M, Nc), jnp.float32),
            pltpu.VMEM((M, Nc), jnp.float32),
            pltpu.SemaphoreType.DMA,
            pltpu.SemaphoreType.DMA,
            pltpu.SemaphoreType.DMA,
            pltpu.SemaphoreType.REGULAR,
        ],
        compiler_params=pltpu.CompilerParams(
            dimension_semantics=("arbitrary",),
            collective_id=0,
            vmem_limit_bytes=100 * 1024 * 1024,
        ),
    )(A, B)


# baseline (device time: 924561 ns/iter reference)
import jax
import jax.numpy as jnp
from jax import lax
from jax.experimental import pallas as pl
from jax.experimental.pallas import tpu as pltpu

M = 4096
K = 2048
N = 4096
NC = 16
Nc = N // NC


def kernel(A, B):
    def body(a_hbm, b_ref, out_ref, a_vmem, send_buf, recv_buf,
             local_sem, send_sem, recv_sem, credit_sem):
        j = pl.program_id(0)
        my_x = lax.axis_index("x")
        my_y = lax.axis_index("y")
        neighbor = (my_x, 1 - my_y)

        @pl.when(j == 0)
        def _():
            cp = pltpu.make_async_copy(a_hbm, a_vmem, local_sem)
            cp.start()
            cp.wait()
            barrier = pltpu.get_barrier_semaphore()
            pl.semaphore_signal(
                barrier, inc=1,
                device_id=neighbor, device_id_type=pl.DeviceIdType.MESH,
            )
            pl.semaphore_wait(barrier, 1)

        send_buf[...] = jnp.dot(
            a_vmem[...], b_ref[...], preferred_element_type=jnp.float32
        )

        @pl.when(j > 0)
        def _():
            pl.semaphore_wait(credit_sem, 1)

        rdma = pltpu.make_async_remote_copy(
            src_ref=send_buf,
            dst_ref=recv_buf,
            send_sem=send_sem,
            recv_sem=recv_sem,
            device_id=neighbor,
            device_id_type=pl.DeviceIdType.MESH,
        )
        rdma.start()
        rdma.wait()

        out_ref[...] = send_buf[...] + recv_buf[...]
        pl.semaphore_signal(
            credit_sem, inc=1,
            device_id=neighbor, device_id_type=pl.DeviceIdType.MESH,
        )

        @pl.when(j == pl.num_programs(0) - 1)
        def _():
            pl.semaphore_wait(credit_sem, 1)

    return pl.pallas_call(
        body,
        grid=(NC,),
        out_shape=jax.ShapeDtypeStruct((M, N), jnp.float32),
        in_specs=[
            pl.BlockSpec(memory_space=pl.ANY),
            pl.BlockSpec((K, Nc), lambda j: (0, j)),
        ],
        out_specs=pl.BlockSpec((M, Nc), lambda j: (0, j)),
        scratch_shapes=[
            pltpu.VMEM((M, K), jnp.bfloat16),
            pltpu.VMEM((M, Nc), jnp.float32),
            pltpu.VMEM((M, Nc), jnp.float32),
            pltpu.SemaphoreType.DMA,
            pltpu.SemaphoreType.DMA,
            pltpu.SemaphoreType.DMA,
            pltpu.SemaphoreType.REGULAR,
        ],
        compiler_params=pltpu.CompilerParams(
            dimension_semantics=("arbitrary",),
            collective_id=0,
            vmem_limit_bytes=100 * 1024 * 1024,
        ),
    )(A.astype(jnp.bfloat16), B.astype(jnp.bfloat16))


# device time: 456112 ns/iter; 2.0270x vs baseline; 2.0270x over previous
import jax
import jax.numpy as jnp
from jax import lax
from jax.experimental import pallas as pl
from jax.experimental.pallas import tpu as pltpu

M = 4096
K = 2048
N = 4096
NC = 16
Nc = N // NC


def kernel(A, B):
    def body(a_hbm, b_ref, out_ref, a_vmem, send_buf, recv_buf,
             local_sem, send_sems, recv_sems, credit_sem):
        j = pl.program_id(0)
        my_x = lax.axis_index("x")
        my_y = lax.axis_index("y")
        neighbor = (my_x, 1 - my_y)

        def slot_rdma(s):
            return pltpu.make_async_remote_copy(
                src_ref=send_buf.at[s],
                dst_ref=recv_buf.at[s],
                send_sem=send_sems.at[s],
                recv_sem=recv_sems.at[s],
                device_id=neighbor,
                device_id_type=pl.DeviceIdType.MESH,
            )

        @pl.when(j == 0)
        def _():
            cp = pltpu.make_async_copy(a_hbm, a_vmem, local_sem)
            cp.start()
            cp.wait()
            barrier = pltpu.get_barrier_semaphore()
            pl.semaphore_signal(
                barrier, inc=1,
                device_id=neighbor, device_id_type=pl.DeviceIdType.MESH,
            )
            pl.semaphore_wait(barrier, 1)

        @pl.when(j < NC)
        def _():
            s = j % 2

            @pl.when(j >= 2)
            def _():
                slot_rdma(s).wait_send()
                pl.semaphore_wait(credit_sem, 1)

            send_buf[s] = jnp.dot(
                a_vmem[...], b_ref[...], preferred_element_type=jnp.float32
            ).astype(jnp.bfloat16)
            slot_rdma(s).start()

        @pl.when(j > 0)
        def _():
            r = (j - 1) % 2
            slot_rdma(r).wait_recv()
            out_ref[...] = (
                send_buf[r].astype(jnp.float32)
                + recv_buf[r].astype(jnp.float32)
            )
            pl.semaphore_signal(
                credit_sem, inc=1,
                device_id=neighbor, device_id_type=pl.DeviceIdType.MESH,
            )

        @pl.when(j == NC)
        def _():
            slot_rdma(0).wait_send()
            slot_rdma(1).wait_send()
            pl.semaphore_wait(credit_sem, 2)

    return pl.pallas_call(
        body,
        grid=(NC + 1,),
        out_shape=jax.ShapeDtypeStruct((M, N), jnp.float32),
        in_specs=[
            pl.BlockSpec(memory_space=pl.ANY),
            pl.BlockSpec((K, Nc), lambda j: (0, jnp.minimum(j, NC - 1))),
        ],
        out_specs=pl.BlockSpec((M, Nc), lambda j: (0, jnp.maximum(j - 1, 0))),
        scratch_shapes=[
            pltpu.VMEM((M, K), jnp.bfloat16),
            pltpu.VMEM((2, M, Nc), jnp.bfloat16),
            pltpu.VMEM((2, M, Nc), jnp.bfloat16),
            pltpu.SemaphoreType.DMA,
            pltpu.SemaphoreType.DMA((2,)),
            pltpu.SemaphoreType.DMA((2,)),
            pltpu.SemaphoreType.REGULAR,
        ],
        compiler_params=pltpu.CompilerParams(
            dimension_semantics=("arbitrary",),
            collective_id=0,
            vmem_limit_bytes=100 * 1024 * 1024,
        ),
    )(A.astype(jnp.bfloat16), B.astype(jnp.bfloat16))
